# Initial kernel scaffold; baseline (speedup 1.0000x reference)
import jax
import jax.numpy as jnp
from jax.experimental import pallas as pl


def kernel(x, edge_index, W1, b1, W2, b2, W_ih, W_hh, b_ih, b_hh, Wfc, bfc):
    raise NotImplementedError("write your pallas kernel here")


# speedup vs baseline: 6.5426x; 6.5426x over previous
import functools

import jax
import jax.numpy as jnp
from jax import lax
from jax.experimental import pallas as pl
from jax.experimental.pallas import tpu as pltpu
from jax.experimental.pallas import tpu_sc as plsc

N = 10000
D = 128
H = 32
T = 10
E = 320000

NC = 2
NS = 16
NW = NC * NS
EPW = E // NW
CH = 80
NCHUNK = EPW // CH
ROWS_PT = 640
NPAD = NS * ROWS_PT


def _mesh():
    return plsc.VectorSubcoreMesh(core_axis_name="c", subcore_axis_name="s")


@functools.partial(
    pl.kernel,
    out_type=jax.ShapeDtypeStruct((NC, NPAD), jnp.float32),
    mesh=_mesh(),
    compiler_params=pltpu.CompilerParams(use_tc_tiling_on_sc=False),
    scratch_types=[
        pltpu.VMEM((NCHUNK, CH), jnp.int32),
        pltpu.VMEM((CH,), jnp.float32),
        pltpu.VMEM((ROWS_PT,), jnp.float32),
        pltpu.VMEM_SHARED((NPAD,), jnp.float32),
        pltpu.SemaphoreType.DMA,
    ],
)
def _sc_degree(dst_hbm, out_hbm, idx_v, ones_v, buf_v, acc_sh, sem):
    cid = lax.axis_index("c")
    sid = lax.axis_index("s")
    wid = sid * NC + cid

    def fill(i, _):
        buf_v[pl.ds(i * 16, 16)] = jnp.zeros((16,), jnp.float32)
        return 0

    lax.fori_loop(0, ROWS_PT // 16, fill, 0)

    def fill1(i, _):
        ones_v[pl.ds(i * 16, 16)] = jnp.ones((16,), jnp.float32)
        return 0

    lax.fori_loop(0, CH // 16, fill1, 0)

    pltpu.sync_copy(buf_v, acc_sh.at[pl.ds(sid * ROWS_PT, ROWS_PT)])
    pltpu.sync_copy(dst_hbm.at[wid], idx_v)
    plsc.subcore_barrier()

    def body(j, _):
        pltpu.sync_copy(ones_v, acc_sh.at[idx_v.at[j]], add=True)
        return 0

    lax.fori_loop(0, NCHUNK, body, 0)
    plsc.subcore_barrier()

    pltpu.sync_copy(acc_sh.at[pl.ds(sid * ROWS_PT, ROWS_PT)], buf_v)
    pltpu.sync_copy(buf_v, out_hbm.at[cid, pl.ds(sid * ROWS_PT, ROWS_PT)])


@functools.partial(
    pl.kernel,
    out_type=jax.ShapeDtypeStruct((NC, NPAD, H), jnp.float32),
    mesh=_mesh(),
    compiler_params=pltpu.CompilerParams(use_tc_tiling_on_sc=False),
    scratch_types=[
        pltpu.VMEM((NCHUNK, CH), jnp.int32),
        pltpu.VMEM((NCHUNK, CH), jnp.int32),
        pltpu.VMEM((CH, H), jnp.float32),
        pltpu.VMEM((ROWS_PT, H), jnp.float32),
        pltpu.VMEM_SHARED((NPAD, H), jnp.float32),
        pltpu.SemaphoreType.DMA,
    ],
)
def _sc_message(src_hbm, dst_hbm, y_hbm, out_hbm, srcv, dstv, rows, buf, acc_sh, sem):
    cid = lax.axis_index("c")
    sid = lax.axis_index("s")
    wid = sid * NC + cid

    def fill(i, _):
        buf[i, pl.ds(0, 16)] = jnp.zeros((16,), jnp.float32)
        buf[i, pl.ds(16, 16)] = jnp.zeros((16,), jnp.float32)
        return 0

    lax.fori_loop(0, ROWS_PT, fill, 0)

    pltpu.sync_copy(buf, acc_sh.at[pl.ds(sid * ROWS_PT, ROWS_PT)])
    pltpu.sync_copy(src_hbm.at[wid], srcv)
    pltpu.sync_copy(dst_hbm.at[wid], dstv)
    plsc.subcore_barrier()

    def body(j, _):
        pltpu.async_copy(y_hbm.at[srcv.at[j]], rows, sem).wait()
        pltpu.sync_copy(rows, acc_sh.at[dstv.at[j]], add=True)
        return 0

    lax.fori_loop(0, NCHUNK, body, 0)
    plsc.subcore_barrier()

    pltpu.sync_copy(acc_sh.at[pl.ds(sid * ROWS_PT, ROWS_PT)], buf)
    pltpu.sync_copy(buf, out_hbm.at[cid, pl.ds(sid * ROWS_PT, ROWS_PT)])


def _tc_first(x, W1, deg0, deg1):
    def body(x_ref, w_ref, d0_ref, d1_ref, y_ref, dinv_ref):
        deg = d0_ref[...] + d1_ref[...] + 1.0
        dinv = lax.rsqrt(deg)
        xw = jnp.dot(x_ref[...], w_ref[...], preferred_element_type=jnp.float32)
        y_ref[...] = xw * dinv
        dinv_ref[...] = dinv

    return pl.pallas_call(
        body,
        out_shape=[
            jax.ShapeDtypeStruct((N, H), jnp.float32),
            jax.ShapeDtypeStruct((N, 1), jnp.float32),
        ],
    )(x, W1, deg0, deg1)


def _tc_mid(a0, a1, y, dinv, b, W2):
    def body(a0_ref, a1_ref, y_ref, dinv_ref, b_ref, w_ref, y2_ref):
        s = a0_ref[...] + a1_ref[...] + y_ref[...]
        h = jnp.maximum(s * dinv_ref[...] + b_ref[...], 0.0)
        hw = jnp.dot(h, w_ref[...], preferred_element_type=jnp.float32)
        y2_ref[...] = hw * dinv_ref[...]

    return pl.pallas_call(
        body,
        out_shape=jax.ShapeDtypeStruct((N, H), jnp.float32),
    )(a0, a1, y, dinv, b, W2)


def _tc_gates(a0, a1, y, dinv, b, W_ihT, bg):
    def body(a0_ref, a1_ref, y_ref, dinv_ref, b_ref, w_ref, bg_ref, g_ref):
        s = a0_ref[...] + a1_ref[...] + y_ref[...]
        h = jnp.maximum(s * dinv_ref[...] + b_ref[...], 0.0)
        g_ref[...] = (
            jnp.dot(h, w_ref[...], preferred_element_type=jnp.float32) + bg_ref[...]
        )

    return pl.pallas_call(
        body,
        out_shape=jax.ShapeDtypeStruct((N, 4 * H), jnp.float32),
    )(a0, a1, y, dinv, b, W_ihT, bg)


def _tc_lstm(G, W_hhT, WfcT, bfc):
    def body(g_ref, whh_ref, wfc_ref, bfc_ref, out_ref, hs_ref):
        whh = whh_ref[...]

        def step(t, carry):
            h, c = carry
            u = jnp.dot(h, whh, preferred_element_type=jnp.float32)
            gates = g_ref[pl.ds(t, 1), :] + u
            i = jax.nn.sigmoid(gates[:, 0:H])
            f = jax.nn.sigmoid(gates[:, H:2 * H])
            g = jnp.tanh(gates[:, 2 * H:3 * H])
            o = jax.nn.sigmoid(gates[:, 3 * H:4 * H])
            c = f * c + i * g
            h = o * jnp.tanh(c)
            hs_ref[pl.ds(t, 1), :] = h
            return (h, c)

        h0 = jnp.zeros((1, H), jnp.float32)
        lax.fori_loop(0, N, step, (h0, h0))
        out_ref[...] = (
            jnp.dot(hs_ref[...], wfc_ref[...], preferred_element_type=jnp.float32)
            + bfc_ref[...]
        )

    return pl.pallas_call(
        body,
        out_shape=jax.ShapeDtypeStruct((N, T), jnp.float32),
        scratch_shapes=[pltpu.VMEM((N, H), jnp.float32)],
    )(G, W_hhT, WfcT, bfc)


def kernel(x, edge_index, W1, b1, W2, b2, W_ih, W_hh, b_ih, b_hh, Wfc, bfc):
    src = edge_index[0].reshape(NW, NCHUNK, CH)
    dst = edge_index[1].reshape(NW, NCHUNK, CH)

    deg_parts = _sc_degree(dst)
    deg0 = deg_parts[0, :N].reshape(N, 1)
    deg1 = deg_parts[1, :N].reshape(N, 1)

    y1, dinv = _tc_first(x, W1, deg0, deg1)

    acc1 = _sc_message(src, dst, y1)
    y2 = _tc_mid(acc1[0, :N], acc1[1, :N], y1, dinv, b1.reshape(1, H), W2)

    acc2 = _sc_message(src, dst, y2)
    G = _tc_gates(
        acc2[0, :N], acc2[1, :N], y2, dinv, b2.reshape(1, H),
        W_ih.T, (b_ih + b_hh).reshape(1, 4 * H),
    )

    return _tc_lstm(G, W_hh.T, Wfc.T, bfc.reshape(1, T))


# speedup vs baseline: 17.9219x; 2.7393x over previous
import functools

import jax
import jax.numpy as jnp
from jax import lax
from jax.experimental import pallas as pl
from jax.experimental.pallas import tpu as pltpu
from jax.experimental.pallas import tpu_sc as plsc

N = 10000
D = 128
H = 32
T = 10
E = 320000

NC = 2
NS = 16
NW = NC * NS
EPW = E // NW
CH = 80
NCHUNK = EPW // CH
ROWS_PT = 640
NPAD = NS * ROWS_PT


def _mesh():
    return plsc.VectorSubcoreMesh(core_axis_name="c", subcore_axis_name="s")


@functools.partial(
    pl.kernel,
    out_type=jax.ShapeDtypeStruct((NC, NPAD), jnp.float32),
    mesh=_mesh(),
    compiler_params=pltpu.CompilerParams(use_tc_tiling_on_sc=False),
    scratch_types=[
        pltpu.VMEM((NCHUNK, CH), jnp.int32),
        pltpu.VMEM((CH,), jnp.float32),
        pltpu.VMEM((ROWS_PT,), jnp.float32),
        pltpu.VMEM_SHARED((NPAD,), jnp.float32),
        pltpu.SemaphoreType.DMA,
    ],
)
def _sc_degree(dst_hbm, out_hbm, idx_v, ones_v, buf_v, acc_sh, sem):
    cid = lax.axis_index("c")
    sid = lax.axis_index("s")
    wid = sid * NC + cid

    def fill(i, _):
        buf_v[pl.ds(i * 16, 16)] = jnp.zeros((16,), jnp.float32)
        return 0

    lax.fori_loop(0, ROWS_PT // 16, fill, 0)

    def fill1(i, _):
        ones_v[pl.ds(i * 16, 16)] = jnp.ones((16,), jnp.float32)
        return 0

    lax.fori_loop(0, CH // 16, fill1, 0)

    pltpu.sync_copy(buf_v, acc_sh.at[pl.ds(sid * ROWS_PT, ROWS_PT)])
    pltpu.sync_copy(dst_hbm.at[wid], idx_v)
    plsc.subcore_barrier()

    def body(j, _):
        pltpu.sync_copy(ones_v, acc_sh.at[idx_v.at[j]], add=True)
        return 0

    lax.fori_loop(0, NCHUNK, body, 0)
    plsc.subcore_barrier()

    pltpu.sync_copy(acc_sh.at[pl.ds(sid * ROWS_PT, ROWS_PT)], buf_v)
    pltpu.sync_copy(buf_v, out_hbm.at[cid, pl.ds(sid * ROWS_PT, ROWS_PT)])


@functools.partial(
    pl.kernel,
    out_type=jax.ShapeDtypeStruct((NC, NPAD, H), jnp.float32),
    mesh=_mesh(),
    compiler_params=pltpu.CompilerParams(use_tc_tiling_on_sc=False),
    scratch_types=[
        pltpu.VMEM((NCHUNK, CH), jnp.int32),
        pltpu.VMEM((NCHUNK, CH), jnp.int32),
        pltpu.VMEM((CH, H), jnp.float32),
        pltpu.VMEM((ROWS_PT, H), jnp.float32),
        pltpu.VMEM_SHARED((NPAD, H), jnp.float32),
        pltpu.SemaphoreType.DMA,
    ],
)
def _sc_message(src_hbm, dst_hbm, y_hbm, out_hbm, srcv, dstv, rows, buf, acc_sh, sem):
    cid = lax.axis_index("c")
    sid = lax.axis_index("s")
    wid = sid * NC + cid

    def fill(i, _):
        buf[i, pl.ds(0, 16)] = jnp.zeros((16,), jnp.float32)
        buf[i, pl.ds(16, 16)] = jnp.zeros((16,), jnp.float32)
        return 0

    lax.fori_loop(0, ROWS_PT, fill, 0)

    pltpu.sync_copy(buf, acc_sh.at[pl.ds(sid * ROWS_PT, ROWS_PT)])
    pltpu.sync_copy(src_hbm.at[wid], srcv)
    pltpu.sync_copy(dst_hbm.at[wid], dstv)
    plsc.subcore_barrier()

    def body(j, _):
        pltpu.async_copy(y_hbm.at[srcv.at[j]], rows, sem).wait()
        pltpu.sync_copy(rows, acc_sh.at[dstv.at[j]], add=True)
        return 0

    lax.fori_loop(0, NCHUNK, body, 0)
    plsc.subcore_barrier()

    pltpu.sync_copy(acc_sh.at[pl.ds(sid * ROWS_PT, ROWS_PT)], buf)
    pltpu.sync_copy(buf, out_hbm.at[cid, pl.ds(sid * ROWS_PT, ROWS_PT)])


def _tc_first(x, W1, deg0, deg1):
    def body(x_ref, w_ref, d0_ref, d1_ref, y_ref, dinv_ref):
        deg = d0_ref[...] + d1_ref[...] + 1.0
        dinv = lax.rsqrt(deg)
        xw = jnp.dot(x_ref[...], w_ref[...], preferred_element_type=jnp.float32)
        y_ref[...] = xw * dinv
        dinv_ref[...] = dinv

    return pl.pallas_call(
        body,
        out_shape=[
            jax.ShapeDtypeStruct((N, H), jnp.float32),
            jax.ShapeDtypeStruct((N, 1), jnp.float32),
        ],
    )(x, W1, deg0, deg1)


def _tc_mid(a0, a1, y, dinv, b, W2):
    def body(a0_ref, a1_ref, y_ref, dinv_ref, b_ref, w_ref, y2_ref):
        s = a0_ref[...] + a1_ref[...] + y_ref[...]
        h = jnp.maximum(s * dinv_ref[...] + b_ref[...], 0.0)
        hw = jnp.dot(h, w_ref[...], preferred_element_type=jnp.float32)
        y2_ref[...] = hw * dinv_ref[...]

    return pl.pallas_call(
        body,
        out_shape=jax.ShapeDtypeStruct((N, H), jnp.float32),
    )(a0, a1, y, dinv, b, W2)


def _tc_gates(a0, a1, y, dinv, b, W_ih4, bg4):
    def body(a0_ref, a1_ref, y_ref, dinv_ref, b_ref, w_ref, bg_ref,
             gi_ref, gf_ref, gg_ref, go_ref):
        s = a0_ref[...] + a1_ref[...] + y_ref[...]
        h = jnp.maximum(s * dinv_ref[...] + b_ref[...], 0.0)
        dn = (((1,), (1,)), ((), ()))
        for k, out in enumerate((gi_ref, gf_ref, gg_ref, go_ref)):
            out[...] = (
                lax.dot_general(h, w_ref[k], dn,
                                preferred_element_type=jnp.float32)
                + bg_ref[k]
            )

    sd = jax.ShapeDtypeStruct((N, H), jnp.float32)
    return pl.pallas_call(
        body,
        out_shape=[sd, sd, sd, sd],
    )(a0, a1, y, dinv, b, W_ih4, bg4)


def _tc_lstm(Gi, Gf, Gg, Go, W_hh4, WfcT, bfc):
    def body(gi_ref, gf_ref, gg_ref, go_ref, whh_ref, wfc_ref, bfc_ref,
             out_ref, hs_ref):
        wi = whh_ref[0]
        wf = whh_ref[1]
        wg = whh_ref[2]
        wo = whh_ref[3]
        dn = (((1,), (1,)), ((), ()))

        def step(t, carry):
            h, c = carry
            ui = lax.dot_general(h, wi, dn, preferred_element_type=jnp.float32)
            uf = lax.dot_general(h, wf, dn, preferred_element_type=jnp.float32)
            ug = lax.dot_general(h, wg, dn, preferred_element_type=jnp.float32)
            uo = lax.dot_general(h, wo, dn, preferred_element_type=jnp.float32)
            row = pl.ds(t, 1)
            i = 0.5 * jnp.tanh(0.5 * (gi_ref[row, :] + ui)) + 0.5
            f = 0.5 * jnp.tanh(0.5 * (gf_ref[row, :] + uf)) + 0.5
            g = jnp.tanh(gg_ref[row, :] + ug)
            o = 0.5 * jnp.tanh(0.5 * (go_ref[row, :] + uo)) + 0.5
            c = f * c + i * g
            h = o * jnp.tanh(c)
            hs_ref[row, :] = h
            return (h, c)

        h0 = jnp.zeros((1, H), jnp.float32)
        lax.fori_loop(0, N, step, (h0, h0))
        out_ref[...] = (
            jnp.dot(hs_ref[...], wfc_ref[...], preferred_element_type=jnp.float32)
            + bfc_ref[...]
        )

    return pl.pallas_call(
        body,
        out_shape=jax.ShapeDtypeStruct((N, T), jnp.float32),
        scratch_shapes=[pltpu.VMEM((N, H), jnp.float32)],
    )(Gi, Gf, Gg, Go, W_hh4, WfcT, bfc)


def kernel(x, edge_index, W1, b1, W2, b2, W_ih, W_hh, b_ih, b_hh, Wfc, bfc):
    src = edge_index[0].reshape(NW, NCHUNK, CH)
    dst = edge_index[1].reshape(NW, NCHUNK, CH)

    deg_parts = _sc_degree(dst)
    deg0 = deg_parts[0, :N].reshape(N, 1)
    deg1 = deg_parts[1, :N].reshape(N, 1)

    y1, dinv = _tc_first(x, W1, deg0, deg1)

    acc1 = _sc_message(src, dst, y1)
    y2 = _tc_mid(acc1[0, :N], acc1[1, :N], y1, dinv, b1.reshape(1, H), W2)

    acc2 = _sc_message(src, dst, y2)
    Gi, Gf, Gg, Go = _tc_gates(
        acc2[0, :N], acc2[1, :N], y2, dinv, b2.reshape(1, H),
        W_ih.reshape(4, H, H), (b_ih + b_hh).reshape(4, 1, H),
    )

    return _tc_lstm(Gi, Gf, Gg, Go, W_hh.reshape(4, H, H), Wfc.T,
                    bfc.reshape(1, T))
